# Initial kernel scaffold; baseline (speedup 1.0000x reference)
import jax
import jax.numpy as jnp
from jax.experimental import pallas as pl


def kernel(x_embed, prompt, prompt_key):
    raise NotImplementedError("write your pallas kernel here")


# speedup vs baseline: 20.8461x; 20.8461x over previous
import jax
import jax.numpy as jnp
from jax.experimental import pallas as pl
from jax.experimental.pallas import tpu as pltpu

B, S, D = 128, 197, 768
P, L, K = 512, 16, 8
BBLK = 8
DT = 128
NSEL = D // DT


def _mean_norm_kernel(x_ref, xn_ref, x2_ref, sgn_ref):
    i = pl.program_id(0)
    xm = jnp.sum(x_ref[...], axis=1) * (1.0 / S)
    ss = jnp.sum(xm * xm, axis=1, keepdims=True)
    xn = xm * jax.lax.rsqrt(jnp.maximum(ss, 1e-12))
    x2_ref[...] = xn
    xn_ref[...] = xn.T.reshape(D, BBLK, 1)
    pos = jnp.sum((xn > 0).astype(jnp.int32), axis=0, keepdims=True)
    neg = jnp.sum((xn < 0).astype(jnp.int32), axis=0, keepdims=True)
    cur = jnp.concatenate([pos, neg], axis=0).T

    @pl.when(i == 0)
    def _():
        sgn_ref[...] = cur

    @pl.when(i != 0)
    def _():
        sgn_ref[...] = sgn_ref[...] + cur


def _pnorm_kernel(pk_ref, pn_ref, pnT_ref):
    pk = pk_ref[...]
    ss = jnp.sum(pk * pk, axis=1, keepdims=True)
    pnorm = pk * jax.lax.rsqrt(jnp.maximum(ss, 1e-12))
    t = pnorm.T
    pnT_ref[...] = t
    pn_ref[...] = t.reshape(D, 1, P)


def _sim_kernel(xcol_ref, pnT_ref, sim_ref):
    sim_ref[...] = (xcol_ref[...] * pnT_ref[...])[None]


def _select_kernel(pnT_ref, sgnT_ref, counts_ref, major_ref):
    i = pl.program_id(0)
    pnT = pnT_ref[...]
    colidx = jax.lax.broadcasted_iota(jnp.int32, (DT, P), 1)

    def top8_mask(a):
        mem = jnp.zeros((DT, P), dtype=jnp.bool_)
        for _ in range(K):
            m = jnp.max(a, axis=1, keepdims=True)
            cand = jnp.where(a == m, colidx, P)
            amin = jnp.min(cand, axis=1, keepdims=True)
            chosen = colidx == amin
            a = jnp.where(chosen, -jnp.inf, a)
            mem = jnp.logical_or(mem, chosen)
        return mem.astype(jnp.int32)

    mem_pos = top8_mask(pnT)
    mem_neg = top8_mask(-pnT)
    npos = sgnT_ref[:, 0:1]
    nneg = sgnT_ref[:, 1:2]
    partial = (jnp.sum(mem_pos * npos, axis=0, keepdims=True)
               + jnp.sum(mem_neg * nneg, axis=0, keepdims=True))
    nzero = B * DT - jnp.sum(sgnT_ref[...])
    pidx = jax.lax.broadcasted_iota(jnp.int32, (1, P), 1)
    partial = partial + jnp.where(pidx < K, nzero, 0)

    prev = counts_ref[...]
    counts = partial + jnp.where(i == 0, 0, prev)
    counts_ref[...] = counts

    @pl.when(i == NSEL - 1)
    def _():
        c = counts
        kidx = jax.lax.broadcasted_iota(jnp.int32, (1, K), 1)
        major = jnp.zeros((1, K), jnp.int32)
        for k in range(K):
            m = jnp.max(c, axis=1, keepdims=True)
            cand = jnp.where(c == m, pidx, P)
            amin = jnp.min(cand, axis=1, keepdims=True)
            major = jnp.where(kidx == k, amin, major)
            c = jnp.where(pidx == amin, -1, c)
        major_ref[...] = major


def _gather_kernel(major_sref, prompt_ref, bp_ref, idx_ref):
    k = pl.program_id(0)
    bp_ref[...] = jnp.broadcast_to(prompt_ref[...], (BBLK, L, D))
    idx_ref[...] = jnp.full((BBLK, 1, L, D), major_sref[k], jnp.int32)


def kernel(x_embed, prompt, prompt_key):
    xn, x2, sgn = pl.pallas_call(
        _mean_norm_kernel,
        grid=(B // BBLK,),
        in_specs=[pl.BlockSpec((BBLK, S, D), lambda i: (i, 0, 0))],
        out_specs=[pl.BlockSpec((D, BBLK, 1), lambda i: (0, i, 0)),
                   pl.BlockSpec((BBLK, D), lambda i: (i, 0)),
                   pl.BlockSpec((D, 2), lambda i: (0, 0))],
        out_shape=[jax.ShapeDtypeStruct((D, B, 1), jnp.float32),
                   jax.ShapeDtypeStruct((B, D), jnp.float32),
                   jax.ShapeDtypeStruct((D, 2), jnp.int32)],
    )(x_embed)

    pn, pnT = pl.pallas_call(
        _pnorm_kernel,
        in_specs=[pl.BlockSpec((P, D), lambda: (0, 0))],
        out_specs=[pl.BlockSpec((D, 1, P), lambda: (0, 0, 0)),
                   pl.BlockSpec((D, P), lambda: (0, 0))],
        out_shape=[jax.ShapeDtypeStruct((D, 1, P), jnp.float32),
                   jax.ShapeDtypeStruct((D, P), jnp.float32)],
    )(prompt_key)

    xcols = x2.reshape(B * D, 1)
    similarity = pl.pallas_call(
        _sim_kernel,
        grid=(B,),
        in_specs=[pl.BlockSpec((D, 1), lambda b: (b, 0)),
                  pl.BlockSpec((D, P), lambda b: (0, 0))],
        out_specs=pl.BlockSpec((1, D, P), lambda b: (b, 0, 0)),
        out_shape=jax.ShapeDtypeStruct((B, D, P), jnp.float32),
    )(xcols, pnT)

    _, major = pl.pallas_call(
        _select_kernel,
        grid=(NSEL,),
        in_specs=[pl.BlockSpec((DT, P), lambda i: (i, 0)),
                  pl.BlockSpec((DT, 2), lambda i: (i, 0))],
        out_specs=[pl.BlockSpec((1, P), lambda i: (0, 0)),
                   pl.BlockSpec((1, K), lambda i: (0, 0))],
        out_shape=[jax.ShapeDtypeStruct((1, P), jnp.int32),
                   jax.ShapeDtypeStruct((1, K), jnp.int32)],
    )(pnT, sgn)

    batched_prompt, idx4 = pl.pallas_call(
        _gather_kernel,
        grid_spec=pltpu.PrefetchScalarGridSpec(
            num_scalar_prefetch=1,
            grid=(K, B // BBLK),
            in_specs=[pl.BlockSpec((1, L, D), lambda k, b, m: (m[k], 0, 0))],
            out_specs=[pl.BlockSpec((BBLK, L, D), lambda k, b, m: (b, k, 0)),
                       pl.BlockSpec((BBLK, 1, L, D),
                                    lambda k, b, m: (b, k, 0, 0))],
        ),
        out_shape=[jax.ShapeDtypeStruct((B, K * L, D), jnp.float32),
                   jax.ShapeDtypeStruct((B, K, L, D), jnp.int32)],
    )(major.reshape(K), prompt)

    return (batched_prompt, similarity, xn, pn, idx4)
